# Initial kernel scaffold; baseline (speedup 1.0000x reference)
import jax
import jax.numpy as jnp
from jax.experimental import pallas as pl


def kernel(features, edges, labels, W1, b1, W2, b2, Wf1, bf1, Wf2, bf2, Wc1, bc1, Wc2, bc2):
    raise NotImplementedError("write your pallas kernel here")


# speedup vs baseline: 4.4541x; 4.4541x over previous
import functools

import jax
import jax.numpy as jnp
from jax import lax
from jax.experimental import pallas as pl
from jax.experimental.pallas import tpu as pltpu
from jax.experimental.pallas import tpu_sc as plsc

GG, NN, EE, FF = 2, 5000, 80000, 128
D1, D2, H1 = 256, 128, 64
AP = 128
CHUNK = 128
NCH = EE // CHUNK
TILES = 16
ROWS_PT = NN // TILES
TAIL = NN - TILES * ROWS_PT
JFULL = (NCH - 1) // TILES
NP = 5120


def _sc_mesh():
    return plsc.VectorSubcoreMesh(core_axis_name="c", subcore_axis_name="s")


def _make_agg(D):
    JJ = D // 16

    def body(table, srcg, dstl, zinit, out, src_v, dst_s, rows_v,
             idx2, pat_v, acc, sem, sem2):
        c = lax.axis_index("c")
        s = lax.axis_index("s")
        r0 = s * ROWS_PT
        base_t = TILES * ROWS_PT

        pltpu.sync_copy(zinit.at[pl.ds(0, ROWS_PT * D)],
                        acc.at[pl.ds(r0 * D, ROWS_PT * D)])

        @pl.when(s == TILES - 1)
        def _():
            pltpu.sync_copy(zinit.at[pl.ds(0, TAIL * D)],
                            acc.at[pl.ds(base_t * D, TAIL * D)])

        for jj in range(JJ):
            pat_v[pl.ds(jj * 16, 16)] = lax.iota(jnp.int32, 16) + jj * 16

        plsc.subcore_barrier()

        nj = JFULL + jnp.where(s == 0, 1, 0)

        def chunk_body(j, carry):
            cid = j * TILES + s
            pltpu.sync_copy(srcg.at[c, cid, 0], src_v)
            pltpu.sync_copy(dstl.at[c, cid, 0], dst_s)
            pltpu.async_copy(table.at[src_v], rows_v, sem).wait()

            def e16_body(e16, carry2):
                dvec = dst_s[pl.ds(e16 * 16, 16)] * D
                descs = []
                for l in range(16):
                    r = e16 * 16 + l
                    base = dvec[l]
                    row = idx2.at[r, 0]
                    for jj in range(JJ):
                        row[pl.ds(jj * 16, 16)] = (
                            pat_v[pl.ds(jj * 16, 16)] + base)
                    descs.append(
                        pltpu.async_copy(rows_v.at[r, 0],
                                         acc.at[idx2.at[r, 0]],
                                         sem2, add=True))
                for dsc in descs:
                    dsc.wait()
                return carry2

            lax.fori_loop(0, CHUNK // 16, e16_body, 0)
            return carry

        lax.fori_loop(0, nj, chunk_body, 0)
        plsc.subcore_barrier()
        pltpu.sync_copy(acc.at[pl.ds(r0 * D, ROWS_PT * D)],
                        out.at[c, 0, pl.ds(r0 * D, ROWS_PT * D)])

        @pl.when(s == TILES - 1)
        def _():
            pltpu.sync_copy(acc.at[pl.ds(base_t * D, TAIL * D)],
                            out.at[c, 0, pl.ds(base_t * D, TAIL * D)])

    scratch = [
        pltpu.VMEM((CHUNK,), jnp.int32),
        pltpu.VMEM((CHUNK,), jnp.int32),
        pltpu.VMEM((CHUNK, 1, D), jnp.float32),
        pltpu.VMEM((CHUNK, 1, D), jnp.int32),
        pltpu.VMEM((D,), jnp.int32),
        pltpu.VMEM_SHARED((NN * D,), jnp.float32),
        pltpu.SemaphoreType.DMA,
        pltpu.SemaphoreType.DMA,
    ]
    out_t = jax.ShapeDtypeStruct((GG, 1, NN * D), jnp.float32)

    return functools.partial(pl.kernel, mesh=_sc_mesh(), out_type=out_t,
                             scratch_types=scratch)(body)


def _make_deg():
    NB = NP // 128

    def body(dstl, ones_hbm, zinit, out, dst_v, ones_v, acc):
        c = lax.axis_index("c")
        s = lax.axis_index("s")
        nb = jnp.where(s < NB - TILES * (NB // TILES), NB // TILES + 1,
                       NB // TILES)

        def init_body(k, carry):
            b = k * TILES + s
            pltpu.sync_copy(zinit, acc.at[pl.ds(b * 128, 128)])
            return carry

        lax.fori_loop(0, nb, init_body, 0)
        pltpu.sync_copy(ones_hbm, ones_v)
        plsc.subcore_barrier()

        nj = JFULL + jnp.where(s == 0, 1, 0)

        def chunk_body(j, carry):
            cid = j * TILES + s
            pltpu.sync_copy(dstl.at[c, cid, 0], dst_v.at[0])
            pltpu.sync_copy(ones_v, acc.at[dst_v.at[0]], add=True)
            return carry

        lax.fori_loop(0, nj, chunk_body, 0)
        plsc.subcore_barrier()

        def out_body(k, carry):
            b = k * TILES + s
            pltpu.sync_copy(acc.at[pl.ds(b * 128, 128)],
                            out.at[c, 0, pl.ds(b * 128, 128)])
            return carry

        lax.fori_loop(0, nb, out_body, 0)

    scratch = [
        pltpu.VMEM((1, CHUNK), jnp.int32),
        pltpu.VMEM((CHUNK,), jnp.float32),
        pltpu.VMEM_SHARED((NP,), jnp.float32),
    ]
    out_t = jax.ShapeDtypeStruct((GG, 1, NP), jnp.float32)
    return functools.partial(pl.kernel, mesh=_sc_mesh(), out_type=out_t,
                             scratch_types=scratch)(body)


def _dinv_of(deg_ref):
    deg = deg_ref[0] + 1.0
    return lax.rsqrt(deg)


def _tc1(degcol, features, W1):
    def body(deg_ref, x_ref, w_ref, o_ref):
        dinv = _dinv_of(deg_ref)
        h = jnp.dot(x_ref[0], w_ref[...], preferred_element_type=jnp.float32)
        o_ref[0] = dinv * h

    return pl.pallas_call(
        body,
        grid=(GG,),
        in_specs=[
            pl.BlockSpec((1, NN, 1), lambda g: (g, 0, 0)),
            pl.BlockSpec((1, NN, FF), lambda g: (g, 0, 0)),
            pl.BlockSpec((FF, D1), lambda g: (0, 0)),
        ],
        out_specs=pl.BlockSpec((1, NN, D1), lambda g: (g, 0, 0)),
        out_shape=jax.ShapeDtypeStruct((GG, NN, D1), jnp.float32),
    )(degcol, features, W1)


def _tc2(degcol, agg1, h1p, W2, b1):
    def body(deg_ref, a_ref, hp_ref, w_ref, b_ref, o_ref):
        dinv = _dinv_of(deg_ref)
        h1 = jnp.maximum(dinv * (a_ref[0] + hp_ref[0]) + b_ref[...], 0.0)
        o_ref[0] = dinv * jnp.dot(h1, w_ref[...],
                                  preferred_element_type=jnp.float32)

    return pl.pallas_call(
        body,
        grid=(GG,),
        in_specs=[
            pl.BlockSpec((1, NN, 1), lambda g: (g, 0, 0)),
            pl.BlockSpec((1, NN, D1), lambda g: (g, 0, 0)),
            pl.BlockSpec((1, NN, D1), lambda g: (g, 0, 0)),
            pl.BlockSpec((D1, D2), lambda g: (0, 0)),
            pl.BlockSpec((1, D1), lambda g: (0, 0)),
        ],
        out_specs=pl.BlockSpec((1, NN, D2), lambda g: (g, 0, 0)),
        out_shape=jax.ShapeDtypeStruct((GG, NN, D2), jnp.float32),
    )(degcol, agg1, h1p, W2, b1)


def _tc3(degcol, agg2, h2p, b2, Wf1, bf1, Wf2P, bf2P):
    def body(deg_ref, a_ref, hp_ref, b2_ref, wf1_ref, bf1_ref, wf2_ref,
             bf2_ref, h2_ref, asn_ref):
        dinv = _dinv_of(deg_ref)
        h2 = dinv * (a_ref[0] + hp_ref[0]) + b2_ref[...]
        h2_ref[0] = h2
        a1 = jnp.tanh(jnp.dot(h2, wf1_ref[...],
                              preferred_element_type=jnp.float32)
                      + bf1_ref[...])
        lg = jnp.dot(a1, wf2_ref[...],
                     preferred_element_type=jnp.float32) + bf2_ref[...]
        m = jnp.max(lg, axis=1, keepdims=True)
        e = jnp.exp(lg - m)
        asn_ref[0] = e / jnp.sum(e, axis=1, keepdims=True)

    return pl.pallas_call(
        body,
        grid=(GG,),
        in_specs=[
            pl.BlockSpec((1, NN, 1), lambda g: (g, 0, 0)),
            pl.BlockSpec((1, NN, D2), lambda g: (g, 0, 0)),
            pl.BlockSpec((1, NN, D2), lambda g: (g, 0, 0)),
            pl.BlockSpec((1, D2), lambda g: (0, 0)),
            pl.BlockSpec((D2, H1), lambda g: (0, 0)),
            pl.BlockSpec((1, H1), lambda g: (0, 0)),
            pl.BlockSpec((H1, AP), lambda g: (0, 0)),
            pl.BlockSpec((1, AP), lambda g: (0, 0)),
        ],
        out_specs=[
            pl.BlockSpec((1, NN, D2), lambda g: (g, 0, 0)),
            pl.BlockSpec((1, NN, AP), lambda g: (g, 0, 0)),
        ],
        out_shape=[
            jax.ShapeDtypeStruct((GG, NN, D2), jnp.float32),
            jax.ShapeDtypeStruct((GG, NN, AP), jnp.float32),
        ],
    )(degcol, agg2, h2p, b2, Wf1, bf1, Wf2P, bf2P)


def _tc4(h2, assignP, aggA, lab, Wc1, bc1, Wc2P, bc2P):
    def body(h2_ref, asn_ref, agg_ref, lab_ref, wc1_ref, bc1_ref, wc2_ref,
             bc2_ref, emb_ref, pos_ref, neg_ref, cls_ref, pen_ref):
        embs, poss, pens = [], [], []
        for g in range(GG):
            A = asn_ref[g]
            H = h2_ref[g]
            Q = agg_ref[g]
            Gm = lax.dot_general(A, H, (((0,), (0,)), ((), ())),
                                 preferred_element_type=jnp.float32)
            NAd = lax.dot_general(Q, A, (((0,), (0,)), ((), ())),
                                  preferred_element_type=jnp.float32)
            na = NAd[0:2, 0:2]
            denom = jnp.maximum(jnp.sum(jnp.abs(na), axis=1, keepdims=True),
                                1e-12)
            ii = lax.broadcasted_iota(jnp.int32, (2, 2), 0)
            jj = lax.broadcasted_iota(jnp.int32, (2, 2), 1)
            diag = jnp.sum(jnp.where(ii == jj, na, 0.0), axis=1,
                           keepdims=True) / denom
            pens.append(jnp.sum((diag - 1.0) ** 2, axis=0,
                                keepdims=True) / 2.0)
            g0 = Gm[0:1, :]
            g1 = Gm[1:2, :]
            emb_ref[pl.ds(g, 1), :] = (g0 + g1) * 0.5
            p = jnp.clip(g0, -100.0, 100.0)
            q = jnp.clip(g1, -100.0, 100.0)
            pos_ref[pl.ds(g, 1), :] = p
            neg_ref[pl.ds(g, 1), :] = q
            embs.append((g0 + g1) * 0.5)
            poss.append(p)
        data = jnp.concatenate(embs + poss, axis=0)
        d1 = jnp.maximum(jnp.dot(data, wc1_ref[...],
                                 preferred_element_type=jnp.float32)
                         + bc1_ref[...], 0.0)
        pr = jnp.maximum(jnp.dot(d1, wc2_ref[...],
                                 preferred_element_type=jnp.float32)
                         + bc2_ref[...], 0.0)
        pr0 = pr[:, 0:1]
        lab2 = jnp.concatenate([lab_ref[...], lab_ref[...]], axis=0)
        cls_ref[...] = jnp.sum((pr0 - lab2) ** 2, axis=0, keepdims=True) / 4.0
        pen_ref[...] = 5.0 * (pens[0] + pens[1]) / 2.0

    return pl.pallas_call(
        body,
        out_shape=[
            jax.ShapeDtypeStruct((GG, D2), jnp.float32),
            jax.ShapeDtypeStruct((GG, D2), jnp.float32),
            jax.ShapeDtypeStruct((GG, D2), jnp.float32),
            jax.ShapeDtypeStruct((1, 1), jnp.float32),
            jax.ShapeDtypeStruct((1, 1), jnp.float32),
        ],
    )(h2, assignP, aggA, lab, Wc1, bc1, Wc2P, bc2P)


_SC_CACHE = {}


def _sc_kernels():
    if not _SC_CACHE:
        _SC_CACHE["agg128"] = _make_agg(D2)
        _SC_CACHE["deg"] = _make_deg()
    return _SC_CACHE


def kernel(features, edges, labels, W1, b1, W2, b2, Wf1, bf1, Wf2, bf2,
           Wc1, bc1, Wc2, bc2):
    edges = edges.astype(jnp.int32)
    src = edges[:, 0, :]
    dst = edges[:, 1, :]
    goff = (jnp.arange(GG, dtype=jnp.int32) * NN)[:, None]
    srcg = (src + goff).reshape(GG, NCH, 1, CHUNK)
    dstl = dst.reshape(GG, NCH, 1, CHUNK)
    onesE = jnp.ones((CHUNK,), jnp.float32)
    zdeg = jnp.zeros((128,), jnp.float32)
    zA = jnp.zeros((ROWS_PT * D2,), jnp.float32)

    sc = _sc_kernels()
    degacc = sc["deg"](dstl, onesE, zdeg)
    degcol = degacc.reshape(GG, NP)[:, :NN].reshape(GG, NN, 1)
    h1p = _tc1(degcol, features, W1)
    s1_halves = []
    for half in range(2):
        hp = h1p[:, :, half * D2:(half + 1) * D2]
        s1h = sc["agg128"](hp.reshape(GG * NN, 1, D2), srcg, dstl, zA)
        s1_halves.append(s1h.reshape(GG, NN, D2))
    agg1 = jnp.concatenate(s1_halves, axis=2)
    h2p = _tc2(degcol, agg1, h1p, W2, b1.reshape(1, D1))
    agg2 = sc["agg128"](h2p.reshape(GG * NN, 1, D2), srcg, dstl,
                        zA).reshape(GG, NN, D2)
    Wf2P = jnp.pad(Wf2, ((0, 0), (0, AP - 2)))
    bf2P = jnp.pad(bf2.reshape(1, 2), ((0, 0), (0, AP - 2)),
                   constant_values=-1e30)
    h2, assignP = _tc3(degcol, agg2, h2p, b2.reshape(1, D2),
                       Wf1, bf1.reshape(1, H1), Wf2P, bf2P)
    aggA = sc["agg128"](assignP.reshape(GG * NN, 1, AP), srcg, dstl, zA)
    Wc2P = jnp.pad(Wc2, ((0, 0), (0, 7)))
    bc2P = jnp.pad(bc2.reshape(1, 1), ((0, 0), (0, 7)))
    emb, pos, neg, cls, pen = _tc4(h2, assignP, aggA.reshape(GG, NN, AP),
                                   labels.reshape(GG, 1), Wc1,
                                   bc1.reshape(1, 64), Wc2P, bc2P)
    return emb, pos, neg, cls[0, 0], pen[0, 0]


# speedup vs baseline: 9.9603x; 2.2362x over previous
import functools

import jax
import jax.numpy as jnp
from jax import lax
from jax.experimental import pallas as pl
from jax.experimental.pallas import tpu as pltpu
from jax.experimental.pallas import tpu_sc as plsc

GG, NN, EE, FF = 2, 5000, 80000, 128
D1, D2, H1 = 256, 128, 64
AP = 128
CHUNK = 128
NCH = EE // CHUNK
TILES = 16
ROWS_PT = NN // TILES
TAIL = NN - TILES * ROWS_PT
JFULL = (NCH - 1) // TILES
NP = 5120


def _sc_mesh():
    return plsc.VectorSubcoreMesh(core_axis_name="c", subcore_axis_name="s")


def _make_agg(D):
    def body(table, srcg, dstl, zinit, out, src_v, dst_s, rows_v, acc,
             sem, sem2):
        c = lax.axis_index("c")
        s = lax.axis_index("s")
        r0 = s * ROWS_PT
        base_t = TILES * ROWS_PT

        pltpu.sync_copy(zinit.at[pl.ds(0, ROWS_PT)],
                        acc.at[pl.ds(r0, ROWS_PT)])

        @pl.when(s == TILES - 1)
        def _():
            pltpu.sync_copy(zinit.at[pl.ds(0, TAIL)],
                            acc.at[pl.ds(base_t, TAIL)])

        plsc.subcore_barrier()

        nj = JFULL + jnp.where(s == 0, 1, 0)

        def chunk_body(j, carry):
            cid = j * TILES + s
            pltpu.sync_copy(srcg.at[c, cid, 0], src_v)
            pltpu.sync_copy(dstl.at[c, cid, 0], dst_s)
            pltpu.async_copy(table.at[src_v], rows_v, sem).wait()
            pltpu.async_copy(rows_v, acc.at[dst_s], sem2, add=True).wait()
            return carry

        lax.fori_loop(0, nj, chunk_body, 0)
        plsc.subcore_barrier()
        pltpu.sync_copy(acc.at[pl.ds(r0, ROWS_PT)],
                        out.at[c, pl.ds(r0, ROWS_PT)])

        @pl.when(s == TILES - 1)
        def _():
            pltpu.sync_copy(acc.at[pl.ds(base_t, TAIL)],
                            out.at[c, pl.ds(base_t, TAIL)])

    scratch = [
        pltpu.VMEM((CHUNK,), jnp.int32),
        pltpu.VMEM((CHUNK,), jnp.int32),
        pltpu.VMEM((CHUNK, D), jnp.float32),
        pltpu.VMEM_SHARED((NN, D), jnp.float32),
        pltpu.SemaphoreType.DMA,
        pltpu.SemaphoreType.DMA,
    ]
    out_t = jax.ShapeDtypeStruct((GG, NN, D), jnp.float32)

    return functools.partial(pl.kernel, mesh=_sc_mesh(), out_type=out_t,
                             scratch_types=scratch)(body)


def _make_deg():
    NB = NP // 128

    def body(dstl, ones_hbm, zinit, out, dst_v, ones_v, acc):
        c = lax.axis_index("c")
        s = lax.axis_index("s")
        nb = jnp.where(s < NB - TILES * (NB // TILES), NB // TILES + 1,
                       NB // TILES)

        def init_body(k, carry):
            b = k * TILES + s
            pltpu.sync_copy(zinit, acc.at[pl.ds(b * 128, 128)])
            return carry

        lax.fori_loop(0, nb, init_body, 0)
        pltpu.sync_copy(ones_hbm, ones_v)
        plsc.subcore_barrier()

        nj = JFULL + jnp.where(s == 0, 1, 0)

        def chunk_body(j, carry):
            cid = j * TILES + s
            pltpu.sync_copy(dstl.at[c, cid, 0], dst_v.at[0])
            pltpu.sync_copy(ones_v, acc.at[dst_v.at[0]], add=True)
            return carry

        lax.fori_loop(0, nj, chunk_body, 0)
        plsc.subcore_barrier()

        def out_body(k, carry):
            b = k * TILES + s
            pltpu.sync_copy(acc.at[pl.ds(b * 128, 128)],
                            out.at[c, 0, pl.ds(b * 128, 128)])
            return carry

        lax.fori_loop(0, nb, out_body, 0)

    scratch = [
        pltpu.VMEM((1, CHUNK), jnp.int32),
        pltpu.VMEM((CHUNK,), jnp.float32),
        pltpu.VMEM_SHARED((NP,), jnp.float32),
    ]
    out_t = jax.ShapeDtypeStruct((GG, 1, NP), jnp.float32)
    return functools.partial(pl.kernel, mesh=_sc_mesh(), out_type=out_t,
                             scratch_types=scratch)(body)


def _dinv_of(deg_ref):
    deg = deg_ref[0] + 1.0
    return lax.rsqrt(deg)


def _tc1(degcol, features, W1):
    def body(deg_ref, x_ref, w_ref, o_ref):
        dinv = _dinv_of(deg_ref)
        h = jnp.dot(x_ref[0], w_ref[...], preferred_element_type=jnp.float32)
        o_ref[0] = dinv * h

    return pl.pallas_call(
        body,
        grid=(GG,),
        in_specs=[
            pl.BlockSpec((1, NN, 1), lambda g: (g, 0, 0)),
            pl.BlockSpec((1, NN, FF), lambda g: (g, 0, 0)),
            pl.BlockSpec((FF, D1), lambda g: (0, 0)),
        ],
        out_specs=pl.BlockSpec((1, NN, D1), lambda g: (g, 0, 0)),
        out_shape=jax.ShapeDtypeStruct((GG, NN, D1), jnp.float32),
    )(degcol, features, W1)


def _tc2(degcol, agg1, h1p, W2, b1):
    def body(deg_ref, a_ref, hp_ref, w_ref, b_ref, o_ref):
        dinv = _dinv_of(deg_ref)
        h1 = jnp.maximum(dinv * (a_ref[0] + hp_ref[0]) + b_ref[...], 0.0)
        o_ref[0] = dinv * jnp.dot(h1, w_ref[...],
                                  preferred_element_type=jnp.float32)

    return pl.pallas_call(
        body,
        grid=(GG,),
        in_specs=[
            pl.BlockSpec((1, NN, 1), lambda g: (g, 0, 0)),
            pl.BlockSpec((1, NN, D1), lambda g: (g, 0, 0)),
            pl.BlockSpec((1, NN, D1), lambda g: (g, 0, 0)),
            pl.BlockSpec((D1, D2), lambda g: (0, 0)),
            pl.BlockSpec((1, D1), lambda g: (0, 0)),
        ],
        out_specs=pl.BlockSpec((1, NN, D2), lambda g: (g, 0, 0)),
        out_shape=jax.ShapeDtypeStruct((GG, NN, D2), jnp.float32),
    )(degcol, agg1, h1p, W2, b1)


def _tc3(degcol, agg2, h2p, b2, Wf1, bf1, Wf2P, bf2P):
    def body(deg_ref, a_ref, hp_ref, b2_ref, wf1_ref, bf1_ref, wf2_ref,
             bf2_ref, h2_ref, asn_ref):
        dinv = _dinv_of(deg_ref)
        h2 = dinv * (a_ref[0] + hp_ref[0]) + b2_ref[...]
        h2_ref[0] = h2
        a1 = jnp.tanh(jnp.dot(h2, wf1_ref[...],
                              preferred_element_type=jnp.float32)
                      + bf1_ref[...])
        lg = jnp.dot(a1, wf2_ref[...],
                     preferred_element_type=jnp.float32) + bf2_ref[...]
        m = jnp.max(lg, axis=1, keepdims=True)
        e = jnp.exp(lg - m)
        asn_ref[0] = e / jnp.sum(e, axis=1, keepdims=True)

    return pl.pallas_call(
        body,
        grid=(GG,),
        in_specs=[
            pl.BlockSpec((1, NN, 1), lambda g: (g, 0, 0)),
            pl.BlockSpec((1, NN, D2), lambda g: (g, 0, 0)),
            pl.BlockSpec((1, NN, D2), lambda g: (g, 0, 0)),
            pl.BlockSpec((1, D2), lambda g: (0, 0)),
            pl.BlockSpec((D2, H1), lambda g: (0, 0)),
            pl.BlockSpec((1, H1), lambda g: (0, 0)),
            pl.BlockSpec((H1, AP), lambda g: (0, 0)),
            pl.BlockSpec((1, AP), lambda g: (0, 0)),
        ],
        out_specs=[
            pl.BlockSpec((1, NN, D2), lambda g: (g, 0, 0)),
            pl.BlockSpec((1, NN, AP), lambda g: (g, 0, 0)),
        ],
        out_shape=[
            jax.ShapeDtypeStruct((GG, NN, D2), jnp.float32),
            jax.ShapeDtypeStruct((GG, NN, AP), jnp.float32),
        ],
    )(degcol, agg2, h2p, b2, Wf1, bf1, Wf2P, bf2P)


def _tc4(h2, assignP, aggA, lab, Wc1, bc1, Wc2P, bc2P):
    def body(h2_ref, asn_ref, agg_ref, lab_ref, wc1_ref, bc1_ref, wc2_ref,
             bc2_ref, emb_ref, pos_ref, neg_ref, cls_ref, pen_ref):
        embs, poss, pens = [], [], []
        for g in range(GG):
            A = asn_ref[g]
            H = h2_ref[g]
            Q = agg_ref[g]
            Gm = lax.dot_general(A, H, (((0,), (0,)), ((), ())),
                                 preferred_element_type=jnp.float32)
            NAd = lax.dot_general(Q, A, (((0,), (0,)), ((), ())),
                                  preferred_element_type=jnp.float32)
            na = NAd[0:2, 0:2]
            denom = jnp.maximum(jnp.sum(jnp.abs(na), axis=1, keepdims=True),
                                1e-12)
            ii = lax.broadcasted_iota(jnp.int32, (2, 2), 0)
            jj = lax.broadcasted_iota(jnp.int32, (2, 2), 1)
            diag = jnp.sum(jnp.where(ii == jj, na, 0.0), axis=1,
                           keepdims=True) / denom
            pens.append(jnp.sum((diag - 1.0) ** 2, axis=0,
                                keepdims=True) / 2.0)
            g0 = Gm[0:1, :]
            g1 = Gm[1:2, :]
            emb_ref[pl.ds(g, 1), :] = (g0 + g1) * 0.5
            p = jnp.clip(g0, -100.0, 100.0)
            q = jnp.clip(g1, -100.0, 100.0)
            pos_ref[pl.ds(g, 1), :] = p
            neg_ref[pl.ds(g, 1), :] = q
            embs.append((g0 + g1) * 0.5)
            poss.append(p)
        data = jnp.concatenate(embs + poss, axis=0)
        d1 = jnp.maximum(jnp.dot(data, wc1_ref[...],
                                 preferred_element_type=jnp.float32)
                         + bc1_ref[...], 0.0)
        pr = jnp.maximum(jnp.dot(d1, wc2_ref[...],
                                 preferred_element_type=jnp.float32)
                         + bc2_ref[...], 0.0)
        pr0 = pr[:, 0:1]
        lab2 = jnp.concatenate([lab_ref[...], lab_ref[...]], axis=0)
        cls_ref[...] = jnp.sum((pr0 - lab2) ** 2, axis=0, keepdims=True) / 4.0
        pen_ref[...] = 5.0 * (pens[0] + pens[1]) / 2.0

    return pl.pallas_call(
        body,
        out_shape=[
            jax.ShapeDtypeStruct((GG, D2), jnp.float32),
            jax.ShapeDtypeStruct((GG, D2), jnp.float32),
            jax.ShapeDtypeStruct((GG, D2), jnp.float32),
            jax.ShapeDtypeStruct((1, 1), jnp.float32),
            jax.ShapeDtypeStruct((1, 1), jnp.float32),
        ],
    )(h2, assignP, aggA, lab, Wc1, bc1, Wc2P, bc2P)


_SC_CACHE = {}


def _sc_kernels():
    if not _SC_CACHE:
        _SC_CACHE["agg128"] = _make_agg(D2)
        _SC_CACHE["deg"] = _make_deg()
    return _SC_CACHE


def kernel(features, edges, labels, W1, b1, W2, b2, Wf1, bf1, Wf2, bf2,
           Wc1, bc1, Wc2, bc2):
    edges = edges.astype(jnp.int32)
    src = edges[:, 0, :]
    dst = edges[:, 1, :]
    goff = (jnp.arange(GG, dtype=jnp.int32) * NN)[:, None]
    srcg = (src + goff).reshape(GG, NCH, 1, CHUNK)
    dstl = dst.reshape(GG, NCH, 1, CHUNK)
    onesE = jnp.ones((CHUNK,), jnp.float32)
    zdeg = jnp.zeros((128,), jnp.float32)
    zA = jnp.zeros((ROWS_PT, D2), jnp.float32)

    sc = _sc_kernels()
    degacc = sc["deg"](dstl, onesE, zdeg)
    degcol = degacc.reshape(GG, NP)[:, :NN].reshape(GG, NN, 1)
    h1p = _tc1(degcol, features, W1)
    s1_halves = []
    for half in range(2):
        hp = h1p[:, :, half * D2:(half + 1) * D2]
        s1h = sc["agg128"](hp.reshape(GG * NN, D2), srcg, dstl, zA)
        s1_halves.append(s1h)
    agg1 = jnp.concatenate(s1_halves, axis=2)
    h2p = _tc2(degcol, agg1, h1p, W2, b1.reshape(1, D1))
    agg2 = sc["agg128"](h2p.reshape(GG * NN, D2), srcg, dstl, zA)
    Wf2P = jnp.pad(Wf2, ((0, 0), (0, AP - 2)))
    bf2P = jnp.pad(bf2.reshape(1, 2), ((0, 0), (0, AP - 2)),
                   constant_values=-1e30)
    h2, assignP = _tc3(degcol, agg2, h2p, b2.reshape(1, D2),
                       Wf1, bf1.reshape(1, H1), Wf2P, bf2P)
    aggA = sc["agg128"](assignP.reshape(GG * NN, AP), srcg, dstl, zA)
    Wc2P = jnp.pad(Wc2, ((0, 0), (0, 7)))
    bc2P = jnp.pad(bc2.reshape(1, 1), ((0, 0), (0, 7)))
    emb, pos, neg, cls, pen = _tc4(h2, assignP, aggA,
                                   labels.reshape(GG, 1), Wc1,
                                   bc1.reshape(1, 64), Wc2P, bc2P)
    return emb, pos, neg, cls[0, 0], pen[0, 0]


# speedup vs baseline: 12.8242x; 1.2875x over previous
import functools

import jax
import jax.numpy as jnp
from jax import lax
from jax.experimental import pallas as pl
from jax.experimental.pallas import tpu as pltpu
from jax.experimental.pallas import tpu_sc as plsc

GG, NN, EE, FF = 2, 5000, 80000, 128
D1, D2, H1 = 256, 128, 64
AP = 128
CHUNK = 128
NCH = EE // CHUNK
TILES = 16
ROWS_PT = NN // TILES
TAIL = NN - TILES * ROWS_PT
JFULL = (NCH - 1) // TILES
NP = 5120


def _sc_mesh():
    return plsc.VectorSubcoreMesh(core_axis_name="c", subcore_axis_name="s")


def _make_agg(D):
    def body(table, srcg, dstl, zinit, out, src_v, dst_s, rows_v,
             src_v1, dst_s1, rows_v1, acc, sem, sem2, semg1, sems1):
        c = lax.axis_index("c")
        s = lax.axis_index("s")
        r0 = s * ROWS_PT
        base_t = TILES * ROWS_PT

        pltpu.sync_copy(zinit.at[pl.ds(0, ROWS_PT)],
                        acc.at[pl.ds(r0, ROWS_PT)])

        @pl.when(s == TILES - 1)
        def _():
            pltpu.sync_copy(zinit.at[pl.ds(0, TAIL)],
                            acc.at[pl.ds(base_t, TAIL)])

        plsc.subcore_barrier()

        def load_issue(cid, src_b, dst_b, rows_b, semg):
            pltpu.sync_copy(srcg.at[c, cid, 0], src_b)
            pltpu.sync_copy(dstl.at[c, cid, 0], dst_b)
            return pltpu.async_copy(table.at[src_b], rows_b, semg)

        def pair_body(j2, carry):
            cid0 = (2 * j2) * TILES + s
            cid1 = (2 * j2 + 1) * TILES + s
            g0 = load_issue(cid0, src_v, dst_s, rows_v, sem)
            g1 = load_issue(cid1, src_v1, dst_s1, rows_v1, semg1)
            g0.wait()
            s0 = pltpu.async_copy(rows_v, acc.at[dst_s], sem2, add=True)
            g1.wait()
            s1 = pltpu.async_copy(rows_v1, acc.at[dst_s1], sems1, add=True)
            s0.wait()
            s1.wait()
            return carry

        lax.fori_loop(0, JFULL // 2, pair_body, 0)

        def tail_chunk(cid):
            load_issue(cid, src_v, dst_s, rows_v, sem).wait()
            pltpu.async_copy(rows_v, acc.at[dst_s], sem2, add=True).wait()

        tail_chunk((JFULL - 1) * TILES + s)

        @pl.when(s == 0)
        def _():
            tail_chunk(JFULL * TILES)

        plsc.subcore_barrier()
        pltpu.sync_copy(acc.at[pl.ds(r0, ROWS_PT)],
                        out.at[c, pl.ds(r0, ROWS_PT)])

        @pl.when(s == TILES - 1)
        def _():
            pltpu.sync_copy(acc.at[pl.ds(base_t, TAIL)],
                            out.at[c, pl.ds(base_t, TAIL)])

    scratch = [
        pltpu.VMEM((CHUNK,), jnp.int32),
        pltpu.VMEM((CHUNK,), jnp.int32),
        pltpu.VMEM((CHUNK, D), jnp.float32),
        pltpu.VMEM((CHUNK,), jnp.int32),
        pltpu.VMEM((CHUNK,), jnp.int32),
        pltpu.VMEM((CHUNK, D), jnp.float32),
        pltpu.VMEM_SHARED((NN, D), jnp.float32),
        pltpu.SemaphoreType.DMA,
        pltpu.SemaphoreType.DMA,
        pltpu.SemaphoreType.DMA,
        pltpu.SemaphoreType.DMA,
    ]
    out_t = jax.ShapeDtypeStruct((GG, NN, D), jnp.float32)

    return functools.partial(pl.kernel, mesh=_sc_mesh(), out_type=out_t,
                             scratch_types=scratch)(body)


def _make_deg():
    NB = NP // 128

    def body(dstl, ones_hbm, zinit, out, dst_v, ones_v, acc):
        c = lax.axis_index("c")
        s = lax.axis_index("s")
        nb = jnp.where(s < NB - TILES * (NB // TILES), NB // TILES + 1,
                       NB // TILES)

        def init_body(k, carry):
            b = k * TILES + s
            pltpu.sync_copy(zinit, acc.at[pl.ds(b * 128, 128)])
            return carry

        lax.fori_loop(0, nb, init_body, 0)
        pltpu.sync_copy(ones_hbm, ones_v)
        plsc.subcore_barrier()

        nj = JFULL + jnp.where(s == 0, 1, 0)

        def chunk_body(j, carry):
            cid = j * TILES + s
            pltpu.sync_copy(dstl.at[c, cid, 0], dst_v.at[0])
            pltpu.sync_copy(ones_v, acc.at[dst_v.at[0]], add=True)
            return carry

        lax.fori_loop(0, nj, chunk_body, 0)
        plsc.subcore_barrier()

        def out_body(k, carry):
            b = k * TILES + s
            pltpu.sync_copy(acc.at[pl.ds(b * 128, 128)],
                            out.at[c, 0, pl.ds(b * 128, 128)])
            return carry

        lax.fori_loop(0, nb, out_body, 0)

    scratch = [
        pltpu.VMEM((1, CHUNK), jnp.int32),
        pltpu.VMEM((CHUNK,), jnp.float32),
        pltpu.VMEM_SHARED((NP,), jnp.float32),
    ]
    out_t = jax.ShapeDtypeStruct((GG, 1, NP), jnp.float32)
    return functools.partial(pl.kernel, mesh=_sc_mesh(), out_type=out_t,
                             scratch_types=scratch)(body)


def _dinv_of(deg_ref):
    deg = deg_ref[0] + 1.0
    return lax.rsqrt(deg)


def _tc1(degcol, features, W1):
    def body(deg_ref, x_ref, w_ref, o_ref):
        dinv = _dinv_of(deg_ref)
        h = jnp.dot(x_ref[0], w_ref[...], preferred_element_type=jnp.float32)
        o_ref[0] = dinv * h

    return pl.pallas_call(
        body,
        grid=(GG,),
        in_specs=[
            pl.BlockSpec((1, NN, 1), lambda g: (g, 0, 0)),
            pl.BlockSpec((1, NN, FF), lambda g: (g, 0, 0)),
            pl.BlockSpec((FF, D1), lambda g: (0, 0)),
        ],
        out_specs=pl.BlockSpec((1, NN, D1), lambda g: (g, 0, 0)),
        out_shape=jax.ShapeDtypeStruct((GG, NN, D1), jnp.float32),
    )(degcol, features, W1)


def _tc2(degcol, agg1, h1p, W2, b1):
    def body(deg_ref, a_ref, hp_ref, w_ref, b_ref, o_ref):
        dinv = _dinv_of(deg_ref)
        h1 = jnp.maximum(dinv * (a_ref[0] + hp_ref[0]) + b_ref[...], 0.0)
        o_ref[0] = dinv * jnp.dot(h1, w_ref[...],
                                  preferred_element_type=jnp.float32)

    return pl.pallas_call(
        body,
        grid=(GG,),
        in_specs=[
            pl.BlockSpec((1, NN, 1), lambda g: (g, 0, 0)),
            pl.BlockSpec((1, NN, D1), lambda g: (g, 0, 0)),
            pl.BlockSpec((1, NN, D1), lambda g: (g, 0, 0)),
            pl.BlockSpec((D1, D2), lambda g: (0, 0)),
            pl.BlockSpec((1, D1), lambda g: (0, 0)),
        ],
        out_specs=pl.BlockSpec((1, NN, D2), lambda g: (g, 0, 0)),
        out_shape=jax.ShapeDtypeStruct((GG, NN, D2), jnp.float32),
    )(degcol, agg1, h1p, W2, b1)


def _tc3(degcol, agg2, h2p, b2, Wf1, bf1, Wf2P, bf2P):
    def body(deg_ref, a_ref, hp_ref, b2_ref, wf1_ref, bf1_ref, wf2_ref,
             bf2_ref, h2_ref, asn_ref):
        dinv = _dinv_of(deg_ref)
        h2 = dinv * (a_ref[0] + hp_ref[0]) + b2_ref[...]
        h2_ref[0] = h2
        a1 = jnp.tanh(jnp.dot(h2, wf1_ref[...],
                              preferred_element_type=jnp.float32)
                      + bf1_ref[...])
        lg = jnp.dot(a1, wf2_ref[...],
                     preferred_element_type=jnp.float32) + bf2_ref[...]
        m = jnp.max(lg, axis=1, keepdims=True)
        e = jnp.exp(lg - m)
        asn_ref[0] = e / jnp.sum(e, axis=1, keepdims=True)

    return pl.pallas_call(
        body,
        grid=(GG,),
        in_specs=[
            pl.BlockSpec((1, NN, 1), lambda g: (g, 0, 0)),
            pl.BlockSpec((1, NN, D2), lambda g: (g, 0, 0)),
            pl.BlockSpec((1, NN, D2), lambda g: (g, 0, 0)),
            pl.BlockSpec((1, D2), lambda g: (0, 0)),
            pl.BlockSpec((D2, H1), lambda g: (0, 0)),
            pl.BlockSpec((1, H1), lambda g: (0, 0)),
            pl.BlockSpec((H1, AP), lambda g: (0, 0)),
            pl.BlockSpec((1, AP), lambda g: (0, 0)),
        ],
        out_specs=[
            pl.BlockSpec((1, NN, D2), lambda g: (g, 0, 0)),
            pl.BlockSpec((1, NN, AP), lambda g: (g, 0, 0)),
        ],
        out_shape=[
            jax.ShapeDtypeStruct((GG, NN, D2), jnp.float32),
            jax.ShapeDtypeStruct((GG, NN, AP), jnp.float32),
        ],
    )(degcol, agg2, h2p, b2, Wf1, bf1, Wf2P, bf2P)


def _tc4(h2, assignP, aggA, lab, Wc1, bc1, Wc2P, bc2P):
    def body(h2_ref, asn_ref, agg_ref, lab_ref, wc1_ref, bc1_ref, wc2_ref,
             bc2_ref, emb_ref, pos_ref, neg_ref, cls_ref, pen_ref):
        embs, poss, pens = [], [], []
        for g in range(GG):
            A = asn_ref[g]
            H = h2_ref[g]
            Q = agg_ref[g]
            Gm = lax.dot_general(A, H, (((0,), (0,)), ((), ())),
                                 preferred_element_type=jnp.float32)
            NAd = lax.dot_general(Q, A, (((0,), (0,)), ((), ())),
                                  preferred_element_type=jnp.float32)
            na = NAd[0:2, 0:2]
            denom = jnp.maximum(jnp.sum(jnp.abs(na), axis=1, keepdims=True),
                                1e-12)
            ii = lax.broadcasted_iota(jnp.int32, (2, 2), 0)
            jj = lax.broadcasted_iota(jnp.int32, (2, 2), 1)
            diag = jnp.sum(jnp.where(ii == jj, na, 0.0), axis=1,
                           keepdims=True) / denom
            pens.append(jnp.sum((diag - 1.0) ** 2, axis=0,
                                keepdims=True) / 2.0)
            g0 = Gm[0:1, :]
            g1 = Gm[1:2, :]
            emb_ref[pl.ds(g, 1), :] = (g0 + g1) * 0.5
            p = jnp.clip(g0, -100.0, 100.0)
            q = jnp.clip(g1, -100.0, 100.0)
            pos_ref[pl.ds(g, 1), :] = p
            neg_ref[pl.ds(g, 1), :] = q
            embs.append((g0 + g1) * 0.5)
            poss.append(p)
        data = jnp.concatenate(embs + poss, axis=0)
        d1 = jnp.maximum(jnp.dot(data, wc1_ref[...],
                                 preferred_element_type=jnp.float32)
                         + bc1_ref[...], 0.0)
        pr = jnp.maximum(jnp.dot(d1, wc2_ref[...],
                                 preferred_element_type=jnp.float32)
                         + bc2_ref[...], 0.0)
        pr0 = pr[:, 0:1]
        lab2 = jnp.concatenate([lab_ref[...], lab_ref[...]], axis=0)
        cls_ref[...] = jnp.sum((pr0 - lab2) ** 2, axis=0, keepdims=True) / 4.0
        pen_ref[...] = 5.0 * (pens[0] + pens[1]) / 2.0

    return pl.pallas_call(
        body,
        out_shape=[
            jax.ShapeDtypeStruct((GG, D2), jnp.float32),
            jax.ShapeDtypeStruct((GG, D2), jnp.float32),
            jax.ShapeDtypeStruct((GG, D2), jnp.float32),
            jax.ShapeDtypeStruct((1, 1), jnp.float32),
            jax.ShapeDtypeStruct((1, 1), jnp.float32),
        ],
    )(h2, assignP, aggA, lab, Wc1, bc1, Wc2P, bc2P)


_SC_CACHE = {}


def _sc_kernels():
    if not _SC_CACHE:
        _SC_CACHE["agg128"] = _make_agg(D2)
        _SC_CACHE["deg"] = _make_deg()
    return _SC_CACHE


def kernel(features, edges, labels, W1, b1, W2, b2, Wf1, bf1, Wf2, bf2,
           Wc1, bc1, Wc2, bc2):
    edges = edges.astype(jnp.int32)
    src = edges[:, 0, :]
    dst = edges[:, 1, :]
    goff = (jnp.arange(GG, dtype=jnp.int32) * NN)[:, None]
    srcg = (src + goff).reshape(GG, NCH, 1, CHUNK)
    dstl = dst.reshape(GG, NCH, 1, CHUNK)
    onesE = jnp.ones((CHUNK,), jnp.float32)
    zdeg = jnp.zeros((128,), jnp.float32)
    zA = jnp.zeros((ROWS_PT, D2), jnp.float32)

    sc = _sc_kernels()
    degacc = sc["deg"](dstl, onesE, zdeg)
    degcol = degacc.reshape(GG, NP)[:, :NN].reshape(GG, NN, 1)
    h1p = _tc1(degcol, features, W1)
    s1_halves = []
    for half in range(2):
        hp = h1p[:, :, half * D2:(half + 1) * D2]
        s1h = sc["agg128"](hp.reshape(GG * NN, D2), srcg, dstl, zA)
        s1_halves.append(s1h)
    agg1 = jnp.concatenate(s1_halves, axis=2)
    h2p = _tc2(degcol, agg1, h1p, W2, b1.reshape(1, D1))
    agg2 = sc["agg128"](h2p.reshape(GG * NN, D2), srcg, dstl, zA)
    Wf2P = jnp.pad(Wf2, ((0, 0), (0, AP - 2)))
    bf2P = jnp.pad(bf2.reshape(1, 2), ((0, 0), (0, AP - 2)),
                   constant_values=-1e30)
    h2, assignP = _tc3(degcol, agg2, h2p, b2.reshape(1, D2),
                       Wf1, bf1.reshape(1, H1), Wf2P, bf2P)
    aggA = sc["agg128"](assignP.reshape(GG * NN, AP), srcg, dstl, zA)
    Wc2P = jnp.pad(Wc2, ((0, 0), (0, 7)))
    bc2P = jnp.pad(bc2.reshape(1, 1), ((0, 0), (0, 7)))
    emb, pos, neg, cls, pen = _tc4(h2, assignP, aggA,
                                   labels.reshape(GG, 1), Wc1,
                                   bc1.reshape(1, 64), Wc2P, bc2P)
    return emb, pos, neg, cls[0, 0], pen[0, 0]
